# baseline (speedup 1.0000x reference)
import functools

import jax
import jax.numpy as jnp
from jax import lax
from jax.experimental import pallas as pl
from jax.experimental.pallas import tpu as pltpu
from jax.experimental.pallas import tpu_sc as plsc

_N = 5000
_BLK = 1280
_NB = 4
_NP = _NB * _BLK
_THR = 0.5

_NC = 2
_NS = 16
_NW = _NC * _NS
_CHUNK = _NP // _NW
_RCOLS = 80
_ROWS = _CHUNK * 5 // _RCOLS


_SUB = _BLK // 4


def _nms_body(xr, yr, Xr, Yr, xc, yc, Xc, Yc, keep_ref):
    q_lt_p = (lax.broadcasted_iota(jnp.int32, (_SUB, _SUB), 0)
              < lax.broadcasted_iota(jnp.int32, (_SUB, _SUB), 1))

    def iou_mat(off, size, rows):
        rx1, ry1, rx2, ry2, r_area = rows
        cx1 = xc[pl.ds(off, size), :]
        cy1 = yc[pl.ds(off, size), :]
        cx2 = Xc[pl.ds(off, size), :]
        cy2 = Yc[pl.ds(off, size), :]
        c_area = (cx2 - cx1) * (cy2 - cy1)
        xx1 = jnp.maximum(cx1, rx1)
        yy1 = jnp.maximum(cy1, ry1)
        xx2 = jnp.minimum(cx2, rx2)
        yy2 = jnp.minimum(cy2, ry2)
        w = jnp.maximum(xx2 - xx1, 0.0)
        h = jnp.maximum(yy2 - yy1, 0.0)
        inter = w * h
        return inter / (c_area + r_area - inter + 1e-9)

    def row_slices(b, lo, width):
        rx1 = xr[pl.ds(b, 1), lo:lo + width]
        ry1 = yr[pl.ds(b, 1), lo:lo + width]
        rx2 = Xr[pl.ds(b, 1), lo:lo + width]
        ry2 = Yr[pl.ds(b, 1), lo:lo + width]
        return rx1, ry1, rx2, ry2, (rx2 - rx1) * (ry2 - ry1)

    def matvec(v, m):
        return lax.dot_general(v, m, (((1,), (0,)), ((), ())),
                               preferred_element_type=jnp.float32)

    def block_step(b, carry):
        rows_full = row_slices(b, 0, _BLK)

        def cross(j, alive):
            adj = (iou_mat(j * _BLK, _BLK, rows_full) > _THR)
            kprev = keep_ref[pl.ds(j, 1), :]
            supp = matvec(kprev, adj.astype(jnp.float32))
            return jnp.where(supp > 0.0, 0.0, alive)

        base = lax.fori_loop(0, b, cross, jnp.ones((1, _BLK), jnp.float32))

        sub_alive = []
        for u in range(4):
            lo = u * _SUB
            rows_u = row_slices(b, lo, _SUB)
            alive_u = base[:, lo:lo + _SUB]
            for v in range(u):
                adj = (iou_mat(b * _BLK + v * _SUB, _SUB, rows_u) > _THR)
                supp = matvec(sub_alive[v], adj.astype(jnp.float32))
                alive_u = jnp.where(supp > 0.0, 0.0, alive_u)
            adj_self = jnp.where(
                (iou_mat(b * _BLK + lo, _SUB, rows_u) > _THR) & q_lt_p,
                1.0, 0.0)
            base_u = alive_u

            def fix_body(c):
                a, _ = c
                new = jnp.where(matvec(a, adj_self) > 0.0, 0.0, base_u)
                return new, jnp.any(new != a)

            alive_u, _ = lax.while_loop(lambda c: c[1], fix_body,
                                        (base_u, True))
            sub_alive.append(alive_u)
            keep_ref[pl.ds(b, 1), lo:lo + _SUB] = alive_u
        return carry

    lax.fori_loop(0, _NB, block_step, 0)


def _nms_sorted_keep(bp):
    x, y, X, Y = bp[:, 0], bp[:, 1], bp[:, 2], bp[:, 3]
    args = (x.reshape(_NB, _BLK), y.reshape(_NB, _BLK),
            X.reshape(_NB, _BLK), Y.reshape(_NB, _BLK),
            x.reshape(_NP, 1), y.reshape(_NP, 1),
            X.reshape(_NP, 1), Y.reshape(_NP, 1))
    keep = pl.pallas_call(
        _nms_body,
        out_shape=jax.ShapeDtypeStruct((_NB, _BLK), jnp.float32),
    )(*args)
    return keep.reshape(_NP)


def _sc_mask_body(keep, ordp, out, k_v, o_v, sem, sem2):
    wid = lax.axis_index("s") * _NC + lax.axis_index("c")
    base = wid * _CHUNK
    cp_o = pltpu.async_copy(ordp.at[pl.ds(base, _CHUNK)], o_v, sem2)
    cp_k = pltpu.async_copy(keep.at[pl.ds(base, _CHUNK)], k_v, sem)
    cp_o.wait()
    cp_k.wait()
    pltpu.async_copy(k_v, out.at[o_v], sem).wait()


_SC_MASK_CACHE = []


def _sc_mask(keep, ordp):
    if not _SC_MASK_CACHE:
        _SC_MASK_CACHE.append(functools.partial(
            pl.kernel,
            out_type=jax.ShapeDtypeStruct((_NP,), jnp.float32),
            mesh=plsc.VectorSubcoreMesh(core_axis_name="c",
                                        subcore_axis_name="s"),
            scratch_types=[
                pltpu.VMEM((_CHUNK,), jnp.float32),
                pltpu.VMEM((_CHUNK,), jnp.int32),
                pltpu.SemaphoreType.DMA,
                pltpu.SemaphoreType.DMA,
            ],
        )(_sc_mask_body))
    return _SC_MASK_CACHE[0](keep, ordp)


def kernel(boxes, scores):
    order = jnp.argsort(-scores)
    bs = boxes[order]
    bp = jnp.pad(bs, ((0, _NP - _N), (0, 0)))
    keep_sorted = _nms_sorted_keep(bp)
    ordp = jnp.concatenate([order, jnp.arange(_N, _NP)]).astype(jnp.int32)
    mask = _sc_mask(keep_sorted, ordp)[:_N]
    out = jnp.concatenate([boxes * mask[:, None], (scores * mask)[:, None]],
                          axis=1)
    return out


# speedup vs baseline: 1.0898x; 1.0898x over previous
import functools

import jax
import jax.numpy as jnp
from jax import lax
from jax.experimental import pallas as pl
from jax.experimental.pallas import tpu as pltpu
from jax.experimental.pallas import tpu_sc as plsc

_N = 5000
_BLK = 1280
_NB = 4
_NP = _NB * _BLK
_THR = 0.5

_NC = 2
_NS = 16
_NW = _NC * _NS
_CHUNK = _NP // _NW
_RCOLS = 80
_ROWS = _CHUNK * 5 // _RCOLS


def _nms_body(xr, yr, Xr, Yr, xc, yc, Xc, Yc, keep_ref):
    q_lt_p = (lax.broadcasted_iota(jnp.int32, (_BLK, _BLK), 0)
              < lax.broadcasted_iota(jnp.int32, (_BLK, _BLK), 1))

    def iou_mat(off, size, rows):
        rx1, ry1, rx2, ry2, r_area = rows
        cx1 = xc[pl.ds(off, size), :]
        cy1 = yc[pl.ds(off, size), :]
        cx2 = Xc[pl.ds(off, size), :]
        cy2 = Yc[pl.ds(off, size), :]
        c_area = (cx2 - cx1) * (cy2 - cy1)
        xx1 = jnp.maximum(cx1, rx1)
        yy1 = jnp.maximum(cy1, ry1)
        xx2 = jnp.minimum(cx2, rx2)
        yy2 = jnp.minimum(cy2, ry2)
        w = jnp.maximum(xx2 - xx1, 0.0)
        h = jnp.maximum(yy2 - yy1, 0.0)
        inter = w * h
        return inter / (c_area + r_area - inter + 1e-9)

    def row_slices(b, lo, width):
        rx1 = xr[pl.ds(b, 1), lo:lo + width]
        ry1 = yr[pl.ds(b, 1), lo:lo + width]
        rx2 = Xr[pl.ds(b, 1), lo:lo + width]
        ry2 = Yr[pl.ds(b, 1), lo:lo + width]
        return rx1, ry1, rx2, ry2, (rx2 - rx1) * (ry2 - ry1)

    def matvec(v, m):
        return lax.dot_general(v, m, (((1,), (0,)), ((), ())),
                               preferred_element_type=jnp.float32)

    def block_step(b, carry):
        rows_full = row_slices(b, 0, _BLK)

        def cross(j, alive):
            adj = (iou_mat(j * _BLK, _BLK, rows_full) > _THR)
            kprev = keep_ref[pl.ds(j, 1), :]
            supp = matvec(kprev, adj.astype(jnp.float32))
            return jnp.where(supp > 0.0, 0.0, alive)

        base = lax.fori_loop(0, b, cross, jnp.ones((1, _BLK), jnp.float32))

        adj_self = jnp.where(
            (iou_mat(b * _BLK, _BLK, rows_full) > _THR) & q_lt_p, 1.0, 0.0)

        def fix_body(c):
            a, _ = c
            new = jnp.where(matvec(a, adj_self) > 0.0, 0.0, base)
            return new, jnp.any(new != a)

        alive, _ = lax.while_loop(lambda c: c[1], fix_body, (base, True))
        keep_ref[pl.ds(b, 1), :] = alive
        return carry

    lax.fori_loop(0, _NB, block_step, 0)


def _nms_sorted_keep(bp):
    x, y, X, Y = bp[:, 0], bp[:, 1], bp[:, 2], bp[:, 3]
    args = (x.reshape(_NB, _BLK), y.reshape(_NB, _BLK),
            X.reshape(_NB, _BLK), Y.reshape(_NB, _BLK),
            x.reshape(_NP, 1), y.reshape(_NP, 1),
            X.reshape(_NP, 1), Y.reshape(_NP, 1))
    keep = pl.pallas_call(
        _nms_body,
        out_shape=jax.ShapeDtypeStruct((_NB, _BLK), jnp.float32),
    )(*args)
    return keep.reshape(_NP)


def _sc_mask_body(keep, ordp, out, k_v, o_v, sem, sem2):
    wid = lax.axis_index("s") * _NC + lax.axis_index("c")
    base = wid * _CHUNK
    cp_o = pltpu.async_copy(ordp.at[pl.ds(base, _CHUNK)], o_v, sem2)
    cp_k = pltpu.async_copy(keep.at[pl.ds(base, _CHUNK)], k_v, sem)
    cp_o.wait()
    cp_k.wait()
    pltpu.async_copy(k_v, out.at[o_v], sem).wait()


_SC_MASK_CACHE = []


def _sc_mask(keep, ordp):
    if not _SC_MASK_CACHE:
        _SC_MASK_CACHE.append(functools.partial(
            pl.kernel,
            out_type=jax.ShapeDtypeStruct((_NP,), jnp.float32),
            mesh=plsc.VectorSubcoreMesh(core_axis_name="c",
                                        subcore_axis_name="s"),
            scratch_types=[
                pltpu.VMEM((_CHUNK,), jnp.float32),
                pltpu.VMEM((_CHUNK,), jnp.int32),
                pltpu.SemaphoreType.DMA,
                pltpu.SemaphoreType.DMA,
            ],
        )(_sc_mask_body))
    return _SC_MASK_CACHE[0](keep, ordp)


def kernel(boxes, scores):
    order = jnp.argsort(-scores)
    bs = boxes[order]
    bp = jnp.pad(bs, ((0, _NP - _N), (0, 0)))
    keep_sorted = _nms_sorted_keep(bp)
    ordp = jnp.concatenate([order, jnp.arange(_N, _NP)]).astype(jnp.int32)
    mask = _sc_mask(keep_sorted, ordp)[:_N]
    out = jnp.concatenate([boxes * mask[:, None], (scores * mask)[:, None]],
                          axis=1)
    return out
